# Initial kernel scaffold; baseline (speedup 1.0000x reference)
import jax
import jax.numpy as jnp
from jax.experimental import pallas as pl


def kernel(x, edge_index, adj_vals, adjZ_vals, M, AM, W0, b0, W1, b1, W2, b2):
    raise NotImplementedError("write your pallas kernel here")


# speedup vs baseline: 7.6218x; 7.6218x over previous
import functools

import jax
import jax.numpy as jnp
from jax import lax
from jax.experimental import pallas as pl
from jax.experimental.pallas import tpu as pltpu
from jax.experimental.pallas import tpu_sc as plsc

N = 10000
E = 320000
D = 128
H = 16

NC = 2
NS = 16
ROWS_PER_TILE = N // NS
CHUNK = 400


def _sc_mesh():
    return plsc.VectorSubcoreMesh(core_axis_name="c", subcore_axis_name="s")


_SC_PARAMS = pltpu.CompilerParams(use_tc_tiling_on_sc=False,
                                  needs_layout_passes=False)


def _spmm128(y, src, dst, vals, zeros):
    DH = D // NC
    ept = E // NS
    nchunk = ept // CHUNK

    @functools.partial(
        pl.kernel,
        out_type=jax.ShapeDtypeStruct((N, D), jnp.float32),
        mesh=_sc_mesh(),
        compiler_params=_SC_PARAMS,
        scratch_types=dict(
            table=pltpu.VMEM_SHARED((N, DH), jnp.float32),
            acc=pltpu.VMEM_SHARED((N, DH), jnp.float32),
            src_v=pltpu.VMEM((CHUNK,), jnp.int32),
            dst_v=pltpu.VMEM((CHUNK,), jnp.int32),
            val_v=pltpu.VMEM((CHUNK,), jnp.float32),
            rows_v=pltpu.VMEM((CHUNK, DH), jnp.float32),
            gsem=pltpu.SemaphoreType.DMA,
        ),
    )
    def k(y_hbm, src_hbm, dst_hbm, val_hbm, zero_hbm, out_hbm,
          table, acc, src_v, dst_v, val_v, rows_v, gsem):
        c = lax.axis_index("c")
        s = lax.axis_index("s")
        r0 = s * ROWS_PER_TILE
        rsl = pl.ds(r0, ROWS_PER_TILE)
        csl = pl.ds(c * DH, DH)
        pltpu.sync_copy(y_hbm.at[rsl, csl], table.at[rsl])
        pltpu.sync_copy(zero_hbm.at[:, :DH], acc.at[rsl])
        plsc.subcore_barrier()

        ebase = s * ept

        def chunk_body(kk, _):
            e0 = ebase + kk * CHUNK
            esl = pl.ds(e0, CHUNK)
            pltpu.sync_copy(src_hbm.at[esl], src_v)
            pltpu.sync_copy(dst_hbm.at[esl], dst_v)
            pltpu.sync_copy(val_hbm.at[esl], val_v)
            pltpu.async_copy(table.at[src_v], rows_v, gsem).wait()

            def scale_body(i, _):
                v = plsc.load_gather(val_v, [jnp.full((16,), i, jnp.int32)])
                for j in range(DH // 16):
                    sl = pl.ds(j * 16, 16)
                    rows_v[i, sl] = rows_v[i, sl] * v
                return 0

            lax.fori_loop(0, CHUNK, scale_body, 0, unroll=4)
            pltpu.sync_copy(rows_v, acc.at[dst_v], add=True)
            return 0

        lax.fori_loop(0, nchunk, chunk_body, 0)
        plsc.subcore_barrier()
        pltpu.sync_copy(acc.at[rsl], out_hbm.at[rsl, csl])

    return k(y, src, dst, vals, zeros)


def _spmm16(y, src, dst, vals, zeros):
    ept = E // (NC * NS)
    nchunk = ept // CHUNK

    @functools.partial(
        pl.kernel,
        out_type=jax.ShapeDtypeStruct((NC, N, H), jnp.float32),
        mesh=_sc_mesh(),
        compiler_params=_SC_PARAMS,
        scratch_types=dict(
            table=pltpu.VMEM_SHARED((N, H), jnp.float32),
            acc=pltpu.VMEM_SHARED((N, H), jnp.float32),
            src_v=pltpu.VMEM((CHUNK,), jnp.int32),
            dst_v=pltpu.VMEM((CHUNK,), jnp.int32),
            val_v=pltpu.VMEM((CHUNK,), jnp.float32),
            rows_v=pltpu.VMEM((CHUNK, H), jnp.float32),
            gsem=pltpu.SemaphoreType.DMA,
        ),
    )
    def k(y_hbm, src_hbm, dst_hbm, val_hbm, zero_hbm, out_hbm,
          table, acc, src_v, dst_v, val_v, rows_v, gsem):
        c = lax.axis_index("c")
        s = lax.axis_index("s")
        r0 = s * ROWS_PER_TILE
        rsl = pl.ds(r0, ROWS_PER_TILE)
        pltpu.sync_copy(y_hbm.at[rsl], table.at[rsl])
        pltpu.sync_copy(zero_hbm.at[:, :H], acc.at[rsl])
        plsc.subcore_barrier()

        ebase = (c * NS + s) * ept

        def chunk_body(kk, _):
            e0 = ebase + kk * CHUNK
            esl = pl.ds(e0, CHUNK)
            pltpu.sync_copy(src_hbm.at[esl], src_v)
            pltpu.sync_copy(dst_hbm.at[esl], dst_v)
            pltpu.sync_copy(val_hbm.at[esl], val_v)
            pltpu.async_copy(table.at[src_v], rows_v, gsem).wait()

            def scale_body(i, _):
                v = plsc.load_gather(val_v, [jnp.full((16,), i, jnp.int32)])
                rows_v[i, :] = rows_v[i, :] * v
                return 0

            lax.fori_loop(0, CHUNK, scale_body, 0, unroll=8)
            pltpu.sync_copy(rows_v, acc.at[dst_v], add=True)
            return 0

        lax.fori_loop(0, nchunk, chunk_body, 0)
        plsc.subcore_barrier()
        pltpu.sync_copy(acc.at[rsl], out_hbm.at[c, rsl])

    return k(y, src, dst, vals, zeros)


def _dotT(a, w):
    return lax.dot_general(a, w, (((1,), (1,)), ((), ())),
                           preferred_element_type=jnp.float32)


def _tc_scale_kernel(x_ref, m_ref, o_ref):
    o_ref[...] = x_ref[...] * m_ref[...]


def _tc_layer1_kernel(a_ref, am_ref, w0_ref, b0_ref, m_ref, w1_ref, o_ref):
    h = a_ref[...] * am_ref[...]
    h = jnp.maximum(_dotT(h, w0_ref[...]) + b0_ref[...], 0.0)
    o_ref[...] = _dotT(h * m_ref[...], w1_ref[...])


def _tc_layer2_kernel(p0_ref, p1_ref, am_ref, b1_ref, w2_ref, o_ref):
    h = (p0_ref[...] + p1_ref[...]) * am_ref[...] + b1_ref[...]
    h = jnp.maximum(h, 0.0)
    o_ref[...] = _dotT(h, w2_ref[...])


def _tc_final_kernel(p0_ref, p1_ref, b2_ref, o_ref):
    z = p0_ref[...] + p1_ref[...] + b2_ref[...]
    m = jnp.max(z, axis=1, keepdims=True)
    zm = z - m
    lse = jnp.log(jnp.sum(jnp.exp(zm), axis=1, keepdims=True))
    o_ref[...] = zm - lse


def _tc_call(body, out_shape, *args):
    return pl.pallas_call(
        body, out_shape=jax.ShapeDtypeStruct(out_shape, jnp.float32))(*args)


def kernel(x, edge_index, adj_vals, adjZ_vals, M, AM, W0, b0, W1, b1, W2, b2):
    src = edge_index[0].astype(jnp.int32)
    dst = edge_index[1].astype(jnp.int32)
    adj_vals = adj_vals.astype(jnp.float32)
    adjZ_vals = adjZ_vals.astype(jnp.float32)
    b0r = b0.reshape(1, D)
    b1r = b1.reshape(1, H)
    b2r = b2.reshape(1, H)
    zeros = jnp.zeros((ROWS_PER_TILE, D // NC), jnp.float32)

    y0 = _tc_call(_tc_scale_kernel, (N, D), x, M)
    a1 = _spmm128(y0, src, dst, adjZ_vals, zeros)
    t2 = _tc_call(_tc_layer1_kernel, (N, H), a1, AM, W0, b0r, M, W1)
    a2 = _spmm16(t2, src, dst, adjZ_vals, zeros)
    t3 = _tc_call(_tc_layer2_kernel, (N, H), a2[0], a2[1], AM, b1r, W2)
    a3 = _spmm16(t3, src, dst, adj_vals, zeros)
    return _tc_call(_tc_final_kernel, (N, H), a3[0], a3[1], b2r)


# speedup vs baseline: 9.5612x; 1.2545x over previous
import functools

import jax
import jax.numpy as jnp
from jax import lax
from jax.experimental import pallas as pl
from jax.experimental.pallas import tpu as pltpu
from jax.experimental.pallas import tpu_sc as plsc

N = 10000
E = 320000
D = 128
H = 16

NC = 2
NS = 16
ROWS_PER_TILE = N // NS
CHUNK = 400


def _sc_mesh():
    return plsc.VectorSubcoreMesh(core_axis_name="c", subcore_axis_name="s")


_SC_PARAMS = pltpu.CompilerParams(use_tc_tiling_on_sc=False,
                                  needs_layout_passes=False)


def _stage_all(copies):
    for cp in copies:
        cp.wait()


def _edge_pipeline(table, acc, src_all, dst_all, val_all, rows2, gsem,
                   nch, dh):
    def gather(k, b):
        pltpu.async_copy(table.at[src_all.at[k]], rows2.at[b], gsem)

    def half(k, b, nb):
        pltpu.make_async_copy(table.at[src_all.at[k]], rows2.at[b],
                              gsem).wait()

        @pl.when(k + 1 < nch)
        def _():
            gather(k + 1, nb)

        def scale_body(i, _):
            v = plsc.load_gather(
                val_all, [jnp.full((16,), k, jnp.int32),
                          jnp.full((16,), i, jnp.int32)])
            for j in range(dh // 16):
                sl = pl.ds(j * 16, 16)
                rows2[b, i, sl] = rows2[b, i, sl] * v
            return 0

        lax.fori_loop(0, CHUNK, scale_body, 0, unroll=8)
        pltpu.sync_copy(rows2.at[b], acc.at[dst_all.at[k]], add=True)

    gather(0, 0)

    def body(kk, _):
        half(2 * kk, 0, 1)
        half(2 * kk + 1, 1, 0)
        return 0

    lax.fori_loop(0, nch // 2, body, 0)
    if nch % 2:
        half(nch - 1, 0, 1)


def _spmm128(y, src2, dst2, val2, zeros):
    DH = D // NC
    CH = 200
    nch = E // NS // CH

    @functools.partial(
        pl.kernel,
        out_type=jax.ShapeDtypeStruct((N, D), jnp.float32),
        mesh=_sc_mesh(),
        compiler_params=_SC_PARAMS,
        scratch_types=dict(
            table=pltpu.VMEM_SHARED((N, DH), jnp.float32),
            acc=pltpu.VMEM_SHARED((N, DH), jnp.float32),
            src_b=pltpu.VMEM((2, 1, CH), jnp.int32),
            dst_b=pltpu.VMEM((2, 1, CH), jnp.int32),
            val_b=pltpu.VMEM((2, 1, CH), jnp.float32),
            rows2=pltpu.VMEM((2, CH, DH), jnp.float32),
            gsem=pltpu.SemaphoreType.DMA,
            isem=pltpu.SemaphoreType.DMA,
            ssem=pltpu.SemaphoreType.DMA,
        ),
    )
    def k(y_hbm, src_hbm, dst_hbm, val_hbm, zero_hbm, out_hbm,
          table, acc, src_b, dst_b, val_b, rows2, gsem, isem, ssem):
        c = lax.axis_index("c")
        s = lax.axis_index("s")
        rsl = pl.ds(s * ROWS_PER_TILE, ROWS_PER_TILE)
        csl = pl.ds(c * DH, DH)
        kbase = s * nch

        def idx_issue(k_, b):
            ks = pl.ds(kbase + k_, 1)
            pltpu.async_copy(src_hbm.at[ks], src_b.at[b], isem)
            pltpu.async_copy(dst_hbm.at[ks], dst_b.at[b], isem)
            pltpu.async_copy(val_hbm.at[ks], val_b.at[b], isem)

        def idx_wait(k_, b):
            ks = pl.ds(kbase + k_, 1)
            pltpu.make_async_copy(src_hbm.at[ks], src_b.at[b], isem).wait()
            pltpu.make_async_copy(dst_hbm.at[ks], dst_b.at[b], isem).wait()
            pltpu.make_async_copy(val_hbm.at[ks], val_b.at[b], isem).wait()

        def gather(b):
            pltpu.async_copy(table.at[src_b.at[b, 0]], rows2.at[b], gsem)

        def gwait(b):
            pltpu.make_async_copy(table.at[src_b.at[b, 0]], rows2.at[b],
                                  gsem).wait()

        _stage_all([
            pltpu.async_copy(y_hbm.at[rsl, csl], table.at[rsl], ssem),
            pltpu.async_copy(zero_hbm, acc.at[rsl], ssem),
        ])
        idx_issue(0, 0)
        plsc.subcore_barrier()
        idx_wait(0, 0)
        gather(0)
        idx_issue(1, 1)

        def half(k_, b, nb):
            gwait(b)

            @pl.when(k_ + 1 < nch)
            def _():
                idx_wait(k_ + 1, nb)
                gather(nb)

            def scale_body(i, _):
                v = plsc.load_gather(
                    val_b, [jnp.full((16,), b, jnp.int32),
                            jnp.zeros((16,), jnp.int32),
                            jnp.full((16,), i, jnp.int32)])
                for j in range(DH // 16):
                    sl = pl.ds(j * 16, 16)
                    rows2[b, i, sl] = rows2[b, i, sl] * v
                return 0

            lax.fori_loop(0, CH, scale_body, 0, unroll=8)
            pltpu.sync_copy(rows2.at[b], acc.at[dst_b.at[b, 0]], add=True)

            @pl.when(k_ + 2 < nch)
            def _():
                idx_issue(k_ + 2, b)

        def body(kk, _):
            half(2 * kk, 0, 1)
            half(2 * kk + 1, 1, 0)
            return 0

        lax.fori_loop(0, nch // 2, body, 0)
        plsc.subcore_barrier()
        pltpu.sync_copy(acc.at[rsl], out_hbm.at[rsl, csl])

    return k(y, src2, dst2, val2, zeros)


def _spmm16(y, src2, dst2, val2, zeros):
    nch = E // (NC * NS) // CHUNK

    @functools.partial(
        pl.kernel,
        out_type=jax.ShapeDtypeStruct((NC, N, H), jnp.float32),
        mesh=_sc_mesh(),
        compiler_params=_SC_PARAMS,
        scratch_types=dict(
            table=pltpu.VMEM_SHARED((N, H), jnp.float32),
            acc=pltpu.VMEM_SHARED((N, H), jnp.float32),
            src_all=pltpu.VMEM((nch, CHUNK), jnp.int32),
            dst_all=pltpu.VMEM((nch, CHUNK), jnp.int32),
            val_all=pltpu.VMEM((nch, CHUNK), jnp.float32),
            rows2=pltpu.VMEM((2, CHUNK, H), jnp.float32),
            gsem=pltpu.SemaphoreType.DMA,
            ssem=pltpu.SemaphoreType.DMA,
        ),
    )
    def k(y_hbm, src_hbm, dst_hbm, val_hbm, zero_hbm, out_hbm,
          table, acc, src_all, dst_all, val_all, rows2, gsem, ssem):
        c = lax.axis_index("c")
        s = lax.axis_index("s")
        rsl = pl.ds(s * ROWS_PER_TILE, ROWS_PER_TILE)
        ksl = pl.ds((c * NS + s) * nch, nch)
        _stage_all([
            pltpu.async_copy(y_hbm.at[rsl], table.at[rsl], ssem),
            pltpu.async_copy(zero_hbm, acc.at[rsl], ssem),
            pltpu.async_copy(src_hbm.at[ksl], src_all, ssem),
            pltpu.async_copy(dst_hbm.at[ksl], dst_all, ssem),
            pltpu.async_copy(val_hbm.at[ksl], val_all, ssem),
        ])
        plsc.subcore_barrier()
        _edge_pipeline(table, acc, src_all, dst_all, val_all, rows2, gsem,
                       nch, H)
        plsc.subcore_barrier()
        pltpu.sync_copy(acc.at[rsl], out_hbm.at[c, rsl])

    return k(y, src2, dst2, val2, zeros)


def _dotT(a, w):
    return lax.dot_general(a, w, (((1,), (1,)), ((), ())),
                           preferred_element_type=jnp.float32)


def _tc_scale_kernel(x_ref, m_ref, o_ref):
    o_ref[...] = x_ref[...] * m_ref[...]


def _tc_layer1_kernel(a_ref, am_ref, w0_ref, b0_ref, m_ref, w1_ref, o_ref):
    h = a_ref[...] * am_ref[...]
    h = jnp.maximum(_dotT(h, w0_ref[...]) + b0_ref[...], 0.0)
    o_ref[...] = _dotT(h * m_ref[...], w1_ref[...])


def _tc_layer2_kernel(p0_ref, p1_ref, am_ref, b1_ref, w2_ref, o_ref):
    h = (p0_ref[...] + p1_ref[...]) * am_ref[...] + b1_ref[...]
    h = jnp.maximum(h, 0.0)
    o_ref[...] = _dotT(h, w2_ref[...])


def _tc_final_kernel(p0_ref, p1_ref, b2_ref, o_ref):
    z = p0_ref[...] + p1_ref[...] + b2_ref[...]
    m = jnp.max(z, axis=1, keepdims=True)
    zm = z - m
    lse = jnp.log(jnp.sum(jnp.exp(zm), axis=1, keepdims=True))
    o_ref[...] = zm - lse


def _tc_call(body, out_shape, *args):
    return pl.pallas_call(
        body, out_shape=jax.ShapeDtypeStruct(out_shape, jnp.float32))(*args)


def kernel(x, edge_index, adj_vals, adjZ_vals, M, AM, W0, b0, W1, b1, W2, b2):
    src = edge_index[0].astype(jnp.int32)
    dst = edge_index[1].astype(jnp.int32)
    src2 = src.reshape(E // CHUNK, CHUNK)
    dst2 = dst.reshape(E // CHUNK, CHUNK)
    adjv = adj_vals.astype(jnp.float32).reshape(E // CHUNK, CHUNK)
    adjZv = adjZ_vals.astype(jnp.float32).reshape(E // CHUNK, CHUNK)
    src2a = src.reshape(E // 200, 200)
    dst2a = dst.reshape(E // 200, 200)
    adjZva = adjZ_vals.astype(jnp.float32).reshape(E // 200, 200)
    b0r = b0.reshape(1, D)
    b1r = b1.reshape(1, H)
    b2r = b2.reshape(1, H)
    zeros64 = jnp.zeros((ROWS_PER_TILE, D // NC), jnp.float32)
    zeros16 = jnp.zeros((ROWS_PER_TILE, H), jnp.float32)

    y0 = _tc_call(_tc_scale_kernel, (N, D), x, M)
    a1 = _spmm128(y0, src2a, dst2a, adjZva, zeros64)
    t2 = _tc_call(_tc_layer1_kernel, (N, H), a1, AM, W0, b0r, M, W1)
    a2 = _spmm16(t2, src2, dst2, adjZv, zeros16)
    t3 = _tc_call(_tc_layer2_kernel, (N, H), a2[0], a2[1], AM, b1r, W2)
    a3 = _spmm16(t3, src2, dst2, adjv, zeros16)
    return _tc_call(_tc_final_kernel, (N, H), a3[0], a3[1], b2r)


# speedup vs baseline: 14.9244x; 1.5609x over previous
import functools

import jax
import jax.numpy as jnp
from jax import lax
from jax.experimental import pallas as pl
from jax.experimental.pallas import tpu as pltpu
from jax.experimental.pallas import tpu_sc as plsc

N = 10000
E = 320000
D = 128
H = 16

NC = 2
NS = 16
ROWS_PER_TILE = N // NS
CHUNK = 400


def _sc_mesh():
    return plsc.VectorSubcoreMesh(core_axis_name="c", subcore_axis_name="s")


_SC_PARAMS = pltpu.CompilerParams(use_tc_tiling_on_sc=False,
                                  needs_layout_passes=False)


def _stage_all(copies):
    for cp in copies:
        cp.wait()


def _edge_pipeline(table, acc, src_all, dst_all, val_all, rows2, gsem,
                   nch, dh):
    def gather(k, b):
        pltpu.async_copy(table.at[src_all.at[k]], rows2.at[b], gsem)

    def half(k, b, nb):
        pltpu.make_async_copy(table.at[src_all.at[k]], rows2.at[b],
                              gsem).wait()

        @pl.when(k + 1 < nch)
        def _():
            gather(k + 1, nb)

        @plsc.parallel_loop(0, CHUNK, unroll=8)
        def scale_body(i):
            v = plsc.load_gather(
                val_all, [jnp.full((16,), k, jnp.int32),
                          jnp.full((16,), i, jnp.int32)])
            for j in range(dh // 16):
                sl = pl.ds(j * 16, 16)
                rows2[b, i, sl] = rows2[b, i, sl] * v
        pltpu.sync_copy(rows2.at[b], acc.at[dst_all.at[k]], add=True)

    gather(0, 0)

    def body(kk, _):
        half(2 * kk, 0, 1)
        half(2 * kk + 1, 1, 0)
        return 0

    lax.fori_loop(0, nch // 2, body, 0)
    if nch % 2:
        half(nch - 1, 0, 1)


def _spmm128(y, src2, dst2, val2, zeros):
    DH = D // NC
    CH = 200
    nch = E // NS // CH

    @functools.partial(
        pl.kernel,
        out_type=jax.ShapeDtypeStruct((N, D), jnp.float32),
        mesh=_sc_mesh(),
        compiler_params=_SC_PARAMS,
        scratch_types=dict(
            table=pltpu.VMEM_SHARED((N, DH), jnp.float32),
            acc=pltpu.VMEM_SHARED((N, DH), jnp.float32),
            src_b=pltpu.VMEM((2, 1, CH), jnp.int32),
            dst_b=pltpu.VMEM((2, 1, CH), jnp.int32),
            val_b=pltpu.VMEM((2, 1, CH), jnp.float32),
            rows2=pltpu.VMEM((2, CH, DH), jnp.float32),
            gsem=pltpu.SemaphoreType.DMA,
            isem=pltpu.SemaphoreType.DMA,
            ssem=pltpu.SemaphoreType.DMA,
        ),
    )
    def k(y_hbm, src_hbm, dst_hbm, val_hbm, zero_hbm, out_hbm,
          table, acc, src_b, dst_b, val_b, rows2, gsem, isem, ssem):
        c = lax.axis_index("c")
        s = lax.axis_index("s")
        rsl = pl.ds(s * ROWS_PER_TILE, ROWS_PER_TILE)
        csl = pl.ds(c * DH, DH)
        kbase = s * nch

        def idx_issue(k_, b):
            ks = pl.ds(kbase + k_, 1)
            pltpu.async_copy(src_hbm.at[ks], src_b.at[b], isem)
            pltpu.async_copy(dst_hbm.at[ks], dst_b.at[b], isem)
            pltpu.async_copy(val_hbm.at[ks], val_b.at[b], isem)

        def idx_wait(k_, b):
            ks = pl.ds(kbase + k_, 1)
            pltpu.make_async_copy(src_hbm.at[ks], src_b.at[b], isem).wait()
            pltpu.make_async_copy(dst_hbm.at[ks], dst_b.at[b], isem).wait()
            pltpu.make_async_copy(val_hbm.at[ks], val_b.at[b], isem).wait()

        def gather(b):
            pltpu.async_copy(table.at[src_b.at[b, 0]], rows2.at[b], gsem)

        def gwait(b):
            pltpu.make_async_copy(table.at[src_b.at[b, 0]], rows2.at[b],
                                  gsem).wait()

        _stage_all([
            pltpu.async_copy(y_hbm.at[rsl, csl], table.at[rsl], ssem),
            pltpu.async_copy(zero_hbm, acc.at[rsl], ssem),
        ])
        idx_issue(0, 0)
        plsc.subcore_barrier()
        idx_wait(0, 0)
        gather(0)
        idx_issue(1, 1)

        def half(k_, b, nb):
            gwait(b)

            @pl.when(k_ + 1 < nch)
            def _():
                idx_wait(k_ + 1, nb)
                gather(nb)

            @plsc.parallel_loop(0, CH, unroll=8)
            def scale_body(i):
                v = plsc.load_gather(
                    val_b, [jnp.full((16,), b, jnp.int32),
                            jnp.zeros((16,), jnp.int32),
                            jnp.full((16,), i, jnp.int32)])
                for j in range(DH // 16):
                    sl = pl.ds(j * 16, 16)
                    rows2[b, i, sl] = rows2[b, i, sl] * v
            pltpu.sync_copy(rows2.at[b], acc.at[dst_b.at[b, 0]], add=True)

            @pl.when(k_ + 2 < nch)
            def _():
                idx_issue(k_ + 2, b)

        def body(kk, _):
            half(2 * kk, 0, 1)
            half(2 * kk + 1, 1, 0)
            return 0

        lax.fori_loop(0, nch // 2, body, 0)
        plsc.subcore_barrier()
        pltpu.sync_copy(acc.at[rsl], out_hbm.at[rsl, csl])

    return k(y, src2, dst2, val2, zeros)


def _spmm16(y, src2, dst2, val2, zeros):
    nch = E // (NC * NS) // CHUNK

    @functools.partial(
        pl.kernel,
        out_type=jax.ShapeDtypeStruct((NC, N, H), jnp.float32),
        mesh=_sc_mesh(),
        compiler_params=_SC_PARAMS,
        scratch_types=dict(
            table=pltpu.VMEM_SHARED((N, H), jnp.float32),
            acc=pltpu.VMEM_SHARED((N, H), jnp.float32),
            src_all=pltpu.VMEM((nch, CHUNK), jnp.int32),
            dst_all=pltpu.VMEM((nch, CHUNK), jnp.int32),
            val_all=pltpu.VMEM((nch, CHUNK), jnp.float32),
            rows2=pltpu.VMEM((2, CHUNK, H), jnp.float32),
            gsem=pltpu.SemaphoreType.DMA,
            ssem=pltpu.SemaphoreType.DMA,
        ),
    )
    def k(y_hbm, src_hbm, dst_hbm, val_hbm, zero_hbm, out_hbm,
          table, acc, src_all, dst_all, val_all, rows2, gsem, ssem):
        c = lax.axis_index("c")
        s = lax.axis_index("s")
        rsl = pl.ds(s * ROWS_PER_TILE, ROWS_PER_TILE)
        ksl = pl.ds((c * NS + s) * nch, nch)
        _stage_all([
            pltpu.async_copy(y_hbm.at[rsl], table.at[rsl], ssem),
            pltpu.async_copy(zero_hbm, acc.at[rsl], ssem),
            pltpu.async_copy(src_hbm.at[ksl], src_all, ssem),
            pltpu.async_copy(dst_hbm.at[ksl], dst_all, ssem),
            pltpu.async_copy(val_hbm.at[ksl], val_all, ssem),
        ])
        plsc.subcore_barrier()
        _edge_pipeline(table, acc, src_all, dst_all, val_all, rows2, gsem,
                       nch, H)
        plsc.subcore_barrier()
        pltpu.sync_copy(acc.at[rsl], out_hbm.at[c, rsl])

    return k(y, src2, dst2, val2, zeros)


def _dotT(a, w):
    return lax.dot_general(a, w, (((1,), (1,)), ((), ())),
                           preferred_element_type=jnp.float32)


def _tc_scale_kernel(x_ref, m_ref, o_ref):
    o_ref[...] = x_ref[...] * m_ref[...]


def _tc_layer1_kernel(a_ref, am_ref, w0_ref, b0_ref, m_ref, w1_ref, o_ref):
    h = a_ref[...] * am_ref[...]
    h = jnp.maximum(_dotT(h, w0_ref[...]) + b0_ref[...], 0.0)
    o_ref[...] = _dotT(h * m_ref[...], w1_ref[...])


def _tc_layer2_kernel(p0_ref, p1_ref, am_ref, b1_ref, w2_ref, o_ref):
    h = (p0_ref[...] + p1_ref[...]) * am_ref[...] + b1_ref[...]
    h = jnp.maximum(h, 0.0)
    o_ref[...] = _dotT(h, w2_ref[...])


def _tc_final_kernel(p0_ref, p1_ref, b2_ref, o_ref):
    z = p0_ref[...] + p1_ref[...] + b2_ref[...]
    m = jnp.max(z, axis=1, keepdims=True)
    zm = z - m
    lse = jnp.log(jnp.sum(jnp.exp(zm), axis=1, keepdims=True))
    o_ref[...] = zm - lse


def _tc_call(body, out_shape, *args):
    return pl.pallas_call(
        body, out_shape=jax.ShapeDtypeStruct(out_shape, jnp.float32))(*args)


def kernel(x, edge_index, adj_vals, adjZ_vals, M, AM, W0, b0, W1, b1, W2, b2):
    src = edge_index[0].astype(jnp.int32)
    dst = edge_index[1].astype(jnp.int32)
    src2 = src.reshape(E // CHUNK, CHUNK)
    dst2 = dst.reshape(E // CHUNK, CHUNK)
    adjv = adj_vals.astype(jnp.float32).reshape(E // CHUNK, CHUNK)
    adjZv = adjZ_vals.astype(jnp.float32).reshape(E // CHUNK, CHUNK)
    src2a = src.reshape(E // 200, 200)
    dst2a = dst.reshape(E // 200, 200)
    adjZva = adjZ_vals.astype(jnp.float32).reshape(E // 200, 200)
    b0r = b0.reshape(1, D)
    b1r = b1.reshape(1, H)
    b2r = b2.reshape(1, H)
    zeros64 = jnp.zeros((ROWS_PER_TILE, D // NC), jnp.float32)
    zeros16 = jnp.zeros((ROWS_PER_TILE, H), jnp.float32)

    y0 = _tc_call(_tc_scale_kernel, (N, D), x, M)
    a1 = _spmm128(y0, src2a, dst2a, adjZva, zeros64)
    t2 = _tc_call(_tc_layer1_kernel, (N, H), a1, AM, W0, b0r, M, W1)
    a2 = _spmm16(t2, src2, dst2, adjZv, zeros16)
    t3 = _tc_call(_tc_layer2_kernel, (N, H), a2[0], a2[1], AM, b1r, W2)
    a3 = _spmm16(t3, src2, dst2, adjv, zeros16)
    return _tc_call(_tc_final_kernel, (N, H), a3[0], a3[1], b2r)


# speedup vs baseline: 17.0961x; 1.1455x over previous
import functools

import jax
import jax.numpy as jnp
from jax import lax
from jax.experimental import pallas as pl
from jax.experimental.pallas import tpu as pltpu
from jax.experimental.pallas import tpu_sc as plsc

N = 10000
E = 320000
D = 128
H = 16

NC = 2
NS = 16
ROWS_PER_TILE = N // NS
CHUNK = 400


def _sc_mesh():
    return plsc.VectorSubcoreMesh(core_axis_name="c", subcore_axis_name="s")


_SC_PARAMS = pltpu.CompilerParams(use_tc_tiling_on_sc=False,
                                  needs_layout_passes=False)


def _stage_all(copies):
    for cp in copies:
        cp.wait()


def _edge_pipeline(table, acc, src_all, dst_all, val_all, rows2, gsem, ssem2,
                   nch, dh):
    def gather(k, b):
        pltpu.async_copy(table.at[src_all.at[k]], rows2.at[b], gsem)

    def swait(b):
        pltpu.make_async_copy(rows2.at[b], acc.at[dst_all.at[0]],
                              ssem2).wait()

    def half(k, b, nb):
        pltpu.make_async_copy(table.at[src_all.at[k]], rows2.at[b],
                              gsem).wait()

        @pl.when(k + 1 < nch)
        def _():
            @pl.when(k >= 1)
            def _():
                swait(nb)

            gather(k + 1, nb)

        @plsc.parallel_loop(0, CHUNK, unroll=8)
        def scale_body(i):
            v = plsc.load_gather(
                val_all, [jnp.full((16,), k, jnp.int32),
                          jnp.full((16,), i, jnp.int32)])
            for j in range(dh // 16):
                sl = pl.ds(j * 16, 16)
                rows2[b, i, sl] = rows2[b, i, sl] * v
        pltpu.async_copy(rows2.at[b], acc.at[dst_all.at[k]], ssem2, add=True)

    gather(0, 0)

    def body(kk, _):
        half(2 * kk, 0, 1)
        half(2 * kk + 1, 1, 0)
        return 0

    lax.fori_loop(0, nch // 2, body, 0)
    if nch % 2:
        half(nch - 1, 0, 1)
    swait(0)
    swait(1)


def _spmm128(y, src2, dst2, val2, zeros):
    DH = D // NC
    CH = 200
    nch = E // NS // CH

    @functools.partial(
        pl.kernel,
        out_type=jax.ShapeDtypeStruct((N, D), jnp.float32),
        mesh=_sc_mesh(),
        compiler_params=_SC_PARAMS,
        scratch_types=dict(
            table=pltpu.VMEM_SHARED((N, DH), jnp.float32),
            acc=pltpu.VMEM_SHARED((N, DH), jnp.float32),
            src_b=pltpu.VMEM((4, 1, CH), jnp.int32),
            dst_b=pltpu.VMEM((4, 1, CH), jnp.int32),
            val_b=pltpu.VMEM((4, 1, CH), jnp.float32),
            rows2=pltpu.VMEM((2, CH, DH), jnp.float32),
            gsem=pltpu.SemaphoreType.DMA,
            isem=pltpu.SemaphoreType.DMA,
            ssem=pltpu.SemaphoreType.DMA,
            ssem2=pltpu.SemaphoreType.DMA,
        ),
    )
    def k(y_hbm, src_hbm, dst_hbm, val_hbm, zero_hbm, out_hbm,
          table, acc, src_b, dst_b, val_b, rows2, gsem, isem, ssem, ssem2):
        c = lax.axis_index("c")
        s = lax.axis_index("s")
        rsl = pl.ds(s * ROWS_PER_TILE, ROWS_PER_TILE)
        csl = pl.ds(c * DH, DH)
        kbase = s * nch

        def idx_issue(k_, b):
            ks = pl.ds(kbase + k_, 1)
            pltpu.async_copy(src_hbm.at[ks], src_b.at[b], isem)
            pltpu.async_copy(dst_hbm.at[ks], dst_b.at[b], isem)
            pltpu.async_copy(val_hbm.at[ks], val_b.at[b], isem)

        def idx_wait(k_, b):
            ks = pl.ds(kbase + k_, 1)
            pltpu.make_async_copy(src_hbm.at[ks], src_b.at[b], isem).wait()
            pltpu.make_async_copy(dst_hbm.at[ks], dst_b.at[b], isem).wait()
            pltpu.make_async_copy(val_hbm.at[ks], val_b.at[b], isem).wait()

        def gather(sl_, rb):
            pltpu.async_copy(table.at[src_b.at[sl_, 0]], rows2.at[rb], gsem)

        def gwait(sl_, rb):
            pltpu.make_async_copy(table.at[src_b.at[sl_, 0]], rows2.at[rb],
                                  gsem).wait()

        def swait(rb):
            pltpu.make_async_copy(rows2.at[rb], acc.at[dst_b.at[0, 0]],
                                  ssem2).wait()

        _stage_all([
            pltpu.async_copy(y_hbm.at[rsl, csl], table.at[rsl], ssem),
            pltpu.async_copy(zero_hbm, acc.at[rsl], ssem),
        ])
        idx_issue(0, 0)
        idx_issue(1, 1)
        plsc.subcore_barrier()
        idx_wait(0, 0)
        gather(0, 0)

        def half(k_, rb, sl_):
            gwait(sl_, rb)

            @pl.when(k_ + 1 < nch)
            def _():
                idx_wait(k_ + 1, (sl_ + 1) % 4)

                @pl.when(k_ >= 1)
                def _():
                    swait(1 - rb)

                gather((sl_ + 1) % 4, 1 - rb)

                @pl.when(k_ + 2 < nch)
                def _():
                    idx_issue(k_ + 2, (sl_ + 2) % 4)

            @plsc.parallel_loop(0, CH, unroll=8)
            def scale_body(i):
                v = plsc.load_gather(
                    val_b, [jnp.full((16,), sl_, jnp.int32),
                            jnp.zeros((16,), jnp.int32),
                            jnp.full((16,), i, jnp.int32)])
                for j in range(DH // 16):
                    sl = pl.ds(j * 16, 16)
                    rows2[rb, i, sl] = rows2[rb, i, sl] * v
            pltpu.async_copy(rows2.at[rb], acc.at[dst_b.at[sl_, 0]], ssem2,
                             add=True)

        def body(kk, _):
            half(4 * kk, 0, 0)
            half(4 * kk + 1, 1, 1)
            half(4 * kk + 2, 0, 2)
            half(4 * kk + 3, 1, 3)
            return 0

        lax.fori_loop(0, nch // 4, body, 0)
        swait(0)
        swait(1)
        plsc.subcore_barrier()
        pltpu.sync_copy(acc.at[rsl], out_hbm.at[rsl, csl])

    return k(y, src2, dst2, val2, zeros)


def _spmm16(y, src2, dst2, val2, zeros):
    nch = E // (NC * NS) // CHUNK

    @functools.partial(
        pl.kernel,
        out_type=jax.ShapeDtypeStruct((NC, N, H), jnp.float32),
        mesh=_sc_mesh(),
        compiler_params=_SC_PARAMS,
        scratch_types=dict(
            table=pltpu.VMEM_SHARED((N, H), jnp.float32),
            acc=pltpu.VMEM_SHARED((N, H), jnp.float32),
            src_all=pltpu.VMEM((nch, CHUNK), jnp.int32),
            dst_all=pltpu.VMEM((nch, CHUNK), jnp.int32),
            val_all=pltpu.VMEM((nch, CHUNK), jnp.float32),
            rows2=pltpu.VMEM((2, CHUNK, H), jnp.float32),
            gsem=pltpu.SemaphoreType.DMA,
            ssem=pltpu.SemaphoreType.DMA,
            ssem2=pltpu.SemaphoreType.DMA,
        ),
    )
    def k(y_hbm, src_hbm, dst_hbm, val_hbm, zero_hbm, out_hbm,
          table, acc, src_all, dst_all, val_all, rows2, gsem, ssem, ssem2):
        c = lax.axis_index("c")
        s = lax.axis_index("s")
        rsl = pl.ds(s * ROWS_PER_TILE, ROWS_PER_TILE)
        ksl = pl.ds((c * NS + s) * nch, nch)
        _stage_all([
            pltpu.async_copy(y_hbm.at[rsl], table.at[rsl], ssem),
            pltpu.async_copy(zero_hbm, acc.at[rsl], ssem),
            pltpu.async_copy(src_hbm.at[ksl], src_all, ssem),
            pltpu.async_copy(dst_hbm.at[ksl], dst_all, ssem),
            pltpu.async_copy(val_hbm.at[ksl], val_all, ssem),
        ])
        plsc.subcore_barrier()
        _edge_pipeline(table, acc, src_all, dst_all, val_all, rows2, gsem,
                       ssem2, nch, H)
        plsc.subcore_barrier()
        pltpu.sync_copy(acc.at[rsl], out_hbm.at[c, rsl])

    return k(y, src2, dst2, val2, zeros)


def _dotT(a, w):
    return lax.dot_general(a, w, (((1,), (1,)), ((), ())),
                           preferred_element_type=jnp.float32)


def _tc_scale_kernel(x_ref, m_ref, o_ref):
    o_ref[...] = x_ref[...] * m_ref[...]


def _tc_layer1_kernel(a_ref, am_ref, w0_ref, b0_ref, m_ref, w1_ref, o_ref):
    h = a_ref[...] * am_ref[...]
    h = jnp.maximum(_dotT(h, w0_ref[...]) + b0_ref[...], 0.0)
    o_ref[...] = _dotT(h * m_ref[...], w1_ref[...])


def _tc_layer2_kernel(p0_ref, p1_ref, am_ref, b1_ref, w2_ref, o_ref):
    h = (p0_ref[...] + p1_ref[...]) * am_ref[...] + b1_ref[...]
    h = jnp.maximum(h, 0.0)
    o_ref[...] = _dotT(h, w2_ref[...])


def _tc_final_kernel(p0_ref, p1_ref, b2_ref, o_ref):
    z = p0_ref[...] + p1_ref[...] + b2_ref[...]
    m = jnp.max(z, axis=1, keepdims=True)
    zm = z - m
    lse = jnp.log(jnp.sum(jnp.exp(zm), axis=1, keepdims=True))
    o_ref[...] = zm - lse


def _tc_call(body, out_shape, *args):
    return pl.pallas_call(
        body, out_shape=jax.ShapeDtypeStruct(out_shape, jnp.float32))(*args)


def kernel(x, edge_index, adj_vals, adjZ_vals, M, AM, W0, b0, W1, b1, W2, b2):
    src = edge_index[0].astype(jnp.int32)
    dst = edge_index[1].astype(jnp.int32)
    src2 = src.reshape(E // CHUNK, CHUNK)
    dst2 = dst.reshape(E // CHUNK, CHUNK)
    adjv = adj_vals.astype(jnp.float32).reshape(E // CHUNK, CHUNK)
    adjZv = adjZ_vals.astype(jnp.float32).reshape(E // CHUNK, CHUNK)
    src2a = src.reshape(E // 200, 200)
    dst2a = dst.reshape(E // 200, 200)
    adjZva = adjZ_vals.astype(jnp.float32).reshape(E // 200, 200)
    b0r = b0.reshape(1, D)
    b1r = b1.reshape(1, H)
    b2r = b2.reshape(1, H)
    zeros64 = jnp.zeros((ROWS_PER_TILE, D // NC), jnp.float32)
    zeros16 = jnp.zeros((ROWS_PER_TILE, H), jnp.float32)

    y0 = _tc_call(_tc_scale_kernel, (N, D), x, M)
    a1 = _spmm128(y0, src2a, dst2a, adjZva, zeros64)
    t2 = _tc_call(_tc_layer1_kernel, (N, H), a1, AM, W0, b0r, M, W1)
    a2 = _spmm16(t2, src2, dst2, adjZv, zeros16)
    t3 = _tc_call(_tc_layer2_kernel, (N, H), a2[0], a2[1], AM, b1r, W2)
    a3 = _spmm16(t3, src2, dst2, adjv, zeros16)
    return _tc_call(_tc_final_kernel, (N, H), a3[0], a3[1], b2r)
